# baseline (speedup 1.0000x reference)
import functools

import jax
import jax.numpy as jnp
from jax import lax
from jax.experimental import pallas as pl
from jax.experimental.pallas import tpu as pltpu
from jax.experimental.pallas import tpu_sc as plsc

N_NODES = 50000
N_EDGES = 800000
EMB = 64
F32 = jnp.float32

NC = 2
NS = 16
HALF = N_NODES // NC
DUMMY = HALF
TROWS = HALF + 8
DW = 16
BLK = 80
NBLK = N_EDGES // BLK
KMAIN = NBLK // NS - 1
ZCH = 200
NCH = HALF // ZCH
CH_ITERS = -(-NCH // NS)

RB = 2000
GRID = N_NODES // RB

_ndspec = lambda w: pl.BlockSpec((RB, w), lambda i: (i, 0))
_wspec = lambda r, c: pl.BlockSpec((r, c), lambda i: (0, 0))

_SC_PARAMS = pltpu.CompilerParams(needs_layout_passes=False,
                                  use_tc_tiling_on_sc=False)


def _tc_ln(x, g, b):
    m = jnp.mean(x, axis=-1, keepdims=True)
    v = jnp.mean((x - m) * (x - m), axis=-1, keepdims=True)
    return (x - m) * lax.rsqrt(v + 1e-5) * g + b


def _dot(a, b):
    return jnp.dot(a, b, preferred_element_type=F32)


def _embed_body(cons, var, cg, cb, c1w, c1b, c2w, c2b, vg, vb, v1w, v1b,
                v2w, v2b, cout, vout):
    x = _tc_ln(cons[...], cg[...], cb[...])
    x = jnp.tanh(_dot(x, c1w[...]) + c1b[...])
    cout[...] = jnp.tanh(_dot(x, c2w[...]) + c2b[...])
    y = _tc_ln(var[...], vg[...], vb[...])
    y = jnp.tanh(_dot(y, v1w[...]) + v1b[...])
    vout[...] = jnp.tanh(_dot(y, v2w[...]) + v2b[...])


_embed_call = pl.pallas_call(
    _embed_body,
    grid=(GRID,),
    in_specs=[
        _ndspec(5), _ndspec(19),
        _wspec(1, 5), _wspec(1, 5), _wspec(5, EMB), _wspec(1, EMB),
        _wspec(EMB, EMB), _wspec(1, EMB),
        _wspec(1, 19), _wspec(1, 19), _wspec(19, EMB), _wspec(1, EMB),
        _wspec(EMB, EMB), _wspec(1, EMB),
    ],
    out_specs=[_ndspec(EMB), _ndspec(EMB)],
    out_shape=[
        jax.ShapeDtypeStruct((N_NODES, EMB), F32),
        jax.ShapeDtypeStruct((N_NODES, EMB), F32),
    ],
)


def _pre_body(right, left, wfl, bfl, wfr, a_out, b_out):
    a_out[...] = _dot(right[...], wfl[...]) + bfl[...]
    b_out[...] = _dot(left[...], wfr[...])


_pre_call = pl.pallas_call(
    _pre_body,
    grid=(GRID,),
    in_specs=[
        _ndspec(EMB), _ndspec(EMB),
        _wspec(EMB, EMB), _wspec(1, EMB), _wspec(EMB, EMB),
    ],
    out_specs=[_ndspec(EMB), _ndspec(EMB)],
    out_shape=[
        jax.ShapeDtypeStruct((N_NODES, EMB), F32),
        jax.ShapeDtypeStruct((N_NODES, EMB), F32),
    ],
)


def _post_body(s, degt, right, wff, bff, g2, b2, wo1a, wo1b, bo1, wo2, bo2,
               out):
    deg = degt[...][:, 0:1]
    agg = _dot(s[...], wff[...]) + deg * bff[...]
    h = _tc_ln(agg, g2[...], b2[...])
    y = _dot(h, wo1a[...]) + _dot(right[...], wo1b[...]) + bo1[...]
    y = jnp.maximum(y, 0.0)
    out[...] = _dot(y, wo2[...]) + bo2[...]


_post_call = pl.pallas_call(
    _post_body,
    grid=(GRID,),
    in_specs=[
        _ndspec(EMB), _ndspec(DW), _ndspec(EMB),
        _wspec(EMB, EMB), _wspec(1, EMB), _wspec(1, EMB), _wspec(1, EMB),
        _wspec(EMB, EMB), _wspec(EMB, EMB), _wspec(1, EMB),
        _wspec(EMB, EMB), _wspec(1, EMB),
    ],
    out_specs=_ndspec(EMB),
    out_shape=jax.ShapeDtypeStruct((N_NODES, EMB), F32),
)


def _final_body(v, w1, b1, w2, out):
    y = jnp.tanh(_dot(v[...], w1[...]) + b1[...])
    z = _dot(y, w2[...])
    out[...] = jax.nn.sigmoid(z)


_final_call = pl.pallas_call(
    _final_body,
    grid=(GRID,),
    in_specs=[
        _ndspec(EMB),
        _wspec(EMB, EMB), _wspec(1, EMB), _wspec(EMB, 1),
    ],
    out_specs=_ndspec(1),
    out_shape=jax.ShapeDtypeStruct((N_NODES, 1), F32),
)


def _local_indices(dsti, loci, base, row=0, n=BLK):
    for j in range(n // 16):
        d = dsti[row, pl.ds(j * 16, 16)]
        l = d - base
        ok = (l >= 0) & (l < HALF)
        loci[row, pl.ds(j * 16, 16)] = jnp.where(ok, l, DUMMY)


def _zero_chunks(sid, zeros_hbm, table):
    def zero_chunk(i, _):
        c = sid + i * NS

        @pl.when(c < NCH)
        def _():
            pltpu.sync_copy(zeros_hbm, table.at[pl.ds(c * ZCH, ZCH)])
        return 0

    lax.fori_loop(0, CH_ITERS, zero_chunk, 0)


def _copy_out_chunks(sid, base, table, out_hbm):
    def out_chunk(i, _):
        c = sid + i * NS

        @pl.when(c < NCH)
        def _():
            pltpu.sync_copy(table.at[pl.ds(c * ZCH, ZCH)],
                            out_hbm.at[pl.ds(base + c * ZCH, ZCH)])
        return 0

    lax.fori_loop(0, CH_ITERS, out_chunk, 0)


def _edge_compute(arows, brows, trows, g, bb):
    def edge2(e2, _):
        for u in range(2):
            e = e2 * 2 + u
            s0 = arows[e, pl.ds(0, 16)] + brows[e, pl.ds(0, 16)]
            s1 = arows[e, pl.ds(16, 16)] + brows[e, pl.ds(16, 16)]
            s2 = arows[e, pl.ds(32, 16)] + brows[e, pl.ds(32, 16)]
            s3 = arows[e, pl.ds(48, 16)] + brows[e, pl.ds(48, 16)]
            hs = jnp.sum(s0 + s1 + s2 + s3)
            hq = jnp.sum(s0 * s0 + s1 * s1 + s2 * s2 + s3 * s3)
            mean = hs * (1.0 / EMB)
            var = hq * (1.0 / EMB) - mean * mean
            x = jnp.full((16,), var + 1e-5, F32)
            iv = plsc.bitcast(x, jnp.int32)
            iv = 0x5F3759DF - lax.shift_right_logical(iv, 1)
            y = plsc.bitcast(iv, F32)
            xh = x * 0.5
            y = y * (1.5 - xh * y * y)
            y = y * (1.5 - xh * y * y)
            y = y * (1.5 - xh * y * y)
            c2 = mean * y
            for k, s in enumerate((s0, s1, s2, s3)):
                t = jnp.maximum((s * y - c2) * g[k] + bb[k], 0.0)
                trows[e, pl.ds(k * 16, 16)] = t
        return 0

    lax.fori_loop(0, BLK // 2, edge2, 0)


def _edge_body(a_hbm, b_hbm, di_hbm, zeros_hbm, gb_hbm, out_hbm,
               dsij, loci, arows, brows, trows, gbv, table, semA, semB):
    cid = lax.axis_index("c")
    sid = lax.axis_index("s")
    base = cid * HALF

    pltpu.sync_copy(gb_hbm, gbv)
    _zero_chunks(sid, zeros_hbm, table)

    g = [gbv[0, pl.ds(k * 16, 16)] for k in range(4)]
    bb = [gbv[1, pl.ds(k * 16, 16)] for k in range(4)]

    plsc.subcore_barrier()

    def idx_gather_issue(k):
        p = k & 1
        b = sid + k * NS

        @pl.when(b < NBLK)
        def _():
            pltpu.sync_copy(di_hbm.at[b], dsij.at[pl.ds(2 * p, 2)])
            pltpu.async_copy(a_hbm.at[dsij.at[2 * p]],
                             arows.at[pl.ds(p * BLK, BLK)], semA.at[p])
            pltpu.async_copy(b_hbm.at[dsij.at[2 * p + 1]],
                             brows.at[pl.ds(p * BLK, BLK)], semB.at[p])

    def process(k):
        p = k & 1
        b = sid + k * NS

        @pl.when(b < NBLK)
        def _():
            _local_indices(dsij, loci, base, row=2 * p)
            ar = arows.at[pl.ds(p * BLK, BLK)]
            br = brows.at[pl.ds(p * BLK, BLK)]
            pltpu.make_async_copy(a_hbm.at[dsij.at[2 * p]], ar,
                                  semA.at[p]).wait()
            pltpu.make_async_copy(b_hbm.at[dsij.at[2 * p + 1]], br,
                                  semB.at[p]).wait()
            _edge_compute(ar, br, trows, g, bb)
            pltpu.sync_copy(trows, table.at[loci.at[2 * p]], add=True)

    idx_gather_issue(0)

    def main_iter(i, _):
        idx_gather_issue(i + 1)
        process(i)
        return 0

    lax.fori_loop(0, KMAIN + 1, main_iter, 0)

    plsc.subcore_barrier()
    _copy_out_chunks(sid, base, table, out_hbm)


_edge_call = pl.kernel(
    _edge_body,
    out_type=jax.ShapeDtypeStruct((N_NODES, EMB), F32),
    mesh=plsc.VectorSubcoreMesh(core_axis_name="c", subcore_axis_name="s"),
    scratch_types=[
        pltpu.VMEM((4, BLK), jnp.int32),
        pltpu.VMEM((4, BLK), jnp.int32),
        pltpu.VMEM((2 * BLK, EMB), F32),
        pltpu.VMEM((2 * BLK, EMB), F32),
        pltpu.VMEM((BLK, EMB), F32),
        pltpu.VMEM((2, EMB), F32),
        pltpu.VMEM_SHARED((TROWS, EMB), F32),
        pltpu.SemaphoreType.DMA((2,)),
        pltpu.SemaphoreType.DMA((2,)),
    ],
    compiler_params=_SC_PARAMS,
)


BLKD = 640
DSUB = BLKD // BLK
NBLKD = N_EDGES // BLKD
KD = (NBLKD // NS) & ~1


def _deg_body(dst_hbm, zeros_hbm, out_hbm, dsti, loci, ones, table):
    cid = lax.axis_index("c")
    sid = lax.axis_index("s")
    base = cid * HALF

    _zero_chunks(sid, zeros_hbm, table)

    onehot = jnp.where(lax.iota(jnp.int32, 16) == 0, 1.0, 0.0).astype(F32)

    def init_row(r, _):
        ones[r, pl.ds(0, 16)] = onehot
        return 0

    lax.fori_loop(0, BLK, init_row, 0)

    plsc.subcore_barrier()

    def do_block(b):
        pltpu.sync_copy(dst_hbm.at[pl.ds(b, 1)], dsti)
        _local_indices(dsti, loci, base)
        pltpu.sync_copy(ones, table.at[loci.at[0]], add=True)

    def block_iter(i, _):
        b = sid + i * NS

        @pl.when(b < NBLK)
        def _():
            do_block(b)
        return 0

    lax.fori_loop(0, KMAIN + 1, block_iter, 0)

    plsc.subcore_barrier()
    _copy_out_chunks(sid, base, table, out_hbm)


_deg_call = pl.kernel(
    _deg_body,
    out_type=jax.ShapeDtypeStruct((N_NODES, DW), F32),
    mesh=plsc.VectorSubcoreMesh(core_axis_name="c", subcore_axis_name="s"),
    scratch_types=[
        pltpu.VMEM((1, BLK), jnp.int32),
        pltpu.VMEM((1, BLK), jnp.int32),
        pltpu.VMEM((BLK, DW), F32),
        pltpu.VMEM_SHARED((TROWS, DW), F32),
    ],
    compiler_params=_SC_PARAMS,
)


def kernel(constraint_features, edge_indices, edge_features, variable_features,
           params):
    p = params
    ei0 = edge_indices[0]
    ei1 = edge_indices[1]
    zeros64 = jnp.zeros((ZCH, EMB), F32)
    zeros16 = jnp.zeros((ZCH, DW), F32)
    r2 = lambda a: a.reshape(1, -1)

    c, v = _embed_call(
        constraint_features, variable_features,
        r2(p["cons_ln"]["g"]), r2(p["cons_ln"]["b"]),
        p["cons_l1"]["W"], r2(p["cons_l1"]["b"]),
        p["cons_l2"]["W"], r2(p["cons_l2"]["b"]),
        r2(p["var_ln"]["g"]), r2(p["var_ln"]["b"]),
        p["var_l1"]["W"], r2(p["var_l1"]["b"]),
        p["var_l2"]["W"], r2(p["var_l2"]["b"]),
    )

    b_edge = p["edge_ln"]["b"][0]

    def run_edge(pc, left, right, di):
        fe = b_edge * pc["fe"]["W"][0]
        A, B = _pre_call(right, left, pc["fl"]["W"],
                         r2(pc["fl"]["b"] + fe), pc["fr"]["W"])
        gb = jnp.stack([pc["ln1"]["g"], pc["ln1"]["b"]])
        return _edge_call(A, B, di, zeros64, gb)

    def run_post(pc, right, S, degt):
        return _post_call(
            S, degt, right, pc["ff"]["W"], r2(pc["ff"]["b"]),
            r2(pc["ln2"]["g"]), r2(pc["ln2"]["b"]),
            pc["o1"]["W"][:EMB], pc["o1"]["W"][EMB:], r2(pc["o1"]["b"]),
            pc["o2"]["W"], r2(pc["o2"]["b"]),
        )

    def run_conv(pc, left, right, di, degt):
        return run_post(pc, right, run_edge(pc, left, right, di), degt)

    e0b = ei0.reshape(-1, 1, BLK)
    e1b = ei1.reshape(-1, 1, BLK)
    di_vc = jnp.concatenate([e0b, e1b], 1)
    di_cv = jnp.concatenate([e1b, e0b], 1)

    S1 = run_edge(p["v2c"], v, c, di_vc)
    deg_c = _deg_call(ei0.reshape(-1, BLK), zeros16)
    deg_v = _deg_call(ei1.reshape(-1, BLK), zeros16)
    c = run_post(p["v2c"], c, S1, deg_c)

    v = run_conv(p["c2v"], c, v, di_cv, deg_v)
    c = run_conv(p["v2c2"], v, c, di_vc, deg_c)
    v = run_conv(p["c2v2"], c, v, di_cv, deg_v)

    out = _final_call(v, p["out_l1"]["W"], r2(p["out_l1"]["b"]),
                      p["out_l2"]["W"])
    return out[:, 0]


# speedup vs baseline: 2.4764x; 2.4764x over previous
import functools

import jax
import jax.numpy as jnp
from jax import lax
from jax.experimental import pallas as pl
from jax.experimental.pallas import tpu as pltpu
from jax.experimental.pallas import tpu_sc as plsc

N_NODES = 50000
N_EDGES = 800000
EMB = 64
F32 = jnp.float32

NC = 2
NS = 16
HALF = N_NODES // NC
DUMMY = HALF
TROWS = HALF + 8
DW = 16
BLK = 80
NBLK = N_EDGES // BLK
KMAIN = NBLK // NS - 1
ZCH = 200
NCH = HALF // ZCH
CH_ITERS = -(-NCH // NS)

RB = 2000
GRID = N_NODES // RB

_ndspec = lambda w: pl.BlockSpec((RB, w), lambda i: (i, 0))
_wspec = lambda r, c: pl.BlockSpec((r, c), lambda i: (0, 0))

_SC_PARAMS = pltpu.CompilerParams(needs_layout_passes=False,
                                  use_tc_tiling_on_sc=False)


def _tc_ln(x, g, b):
    m = jnp.mean(x, axis=-1, keepdims=True)
    v = jnp.mean((x - m) * (x - m), axis=-1, keepdims=True)
    return (x - m) * lax.rsqrt(v + 1e-5) * g + b


def _dot(a, b):
    return jnp.dot(a, b, preferred_element_type=F32)


def _embed_body(cons, var, cg, cb, c1w, c1b, c2w, c2b, vg, vb, v1w, v1b,
                v2w, v2b, cout, vout):
    x = _tc_ln(cons[...], cg[...], cb[...])
    x = jnp.tanh(_dot(x, c1w[...]) + c1b[...])
    cout[...] = jnp.tanh(_dot(x, c2w[...]) + c2b[...])
    y = _tc_ln(var[...], vg[...], vb[...])
    y = jnp.tanh(_dot(y, v1w[...]) + v1b[...])
    vout[...] = jnp.tanh(_dot(y, v2w[...]) + v2b[...])


_embed_call = pl.pallas_call(
    _embed_body,
    grid=(GRID,),
    in_specs=[
        _ndspec(5), _ndspec(19),
        _wspec(1, 5), _wspec(1, 5), _wspec(5, EMB), _wspec(1, EMB),
        _wspec(EMB, EMB), _wspec(1, EMB),
        _wspec(1, 19), _wspec(1, 19), _wspec(19, EMB), _wspec(1, EMB),
        _wspec(EMB, EMB), _wspec(1, EMB),
    ],
    out_specs=[_ndspec(EMB), _ndspec(EMB)],
    out_shape=[
        jax.ShapeDtypeStruct((N_NODES, EMB), F32),
        jax.ShapeDtypeStruct((N_NODES, EMB), F32),
    ],
)


def _pre_body(right, left, wfl, bfl, wfr, a_out, b_out):
    a_out[...] = _dot(right[...], wfl[...]) + bfl[...]
    b_out[...] = _dot(left[...], wfr[...])


_pre_call = pl.pallas_call(
    _pre_body,
    grid=(GRID,),
    in_specs=[
        _ndspec(EMB), _ndspec(EMB),
        _wspec(EMB, EMB), _wspec(1, EMB), _wspec(EMB, EMB),
    ],
    out_specs=[_ndspec(EMB), _ndspec(EMB)],
    out_shape=[
        jax.ShapeDtypeStruct((N_NODES, EMB), F32),
        jax.ShapeDtypeStruct((N_NODES, EMB), F32),
    ],
)


def _post_body(s, degt, right, wff, bff, g2, b2, wo1a, wo1b, bo1, wo2, bo2,
               out):
    deg = degt[...][:, 0:1]
    agg = _dot(s[...], wff[...]) + deg * bff[...]
    h = _tc_ln(agg, g2[...], b2[...])
    y = _dot(h, wo1a[...]) + _dot(right[...], wo1b[...]) + bo1[...]
    y = jnp.maximum(y, 0.0)
    out[...] = _dot(y, wo2[...]) + bo2[...]


_post_call = pl.pallas_call(
    _post_body,
    grid=(GRID,),
    in_specs=[
        _ndspec(EMB), _ndspec(DW), _ndspec(EMB),
        _wspec(EMB, EMB), _wspec(1, EMB), _wspec(1, EMB), _wspec(1, EMB),
        _wspec(EMB, EMB), _wspec(EMB, EMB), _wspec(1, EMB),
        _wspec(EMB, EMB), _wspec(1, EMB),
    ],
    out_specs=_ndspec(EMB),
    out_shape=jax.ShapeDtypeStruct((N_NODES, EMB), F32),
)


def _final_body(v, w1, b1, w2, out):
    y = jnp.tanh(_dot(v[...], w1[...]) + b1[...])
    z = _dot(y, w2[...])
    out[...] = jax.nn.sigmoid(z)


_final_call = pl.pallas_call(
    _final_body,
    grid=(GRID,),
    in_specs=[
        _ndspec(EMB),
        _wspec(EMB, EMB), _wspec(1, EMB), _wspec(EMB, 1),
    ],
    out_specs=_ndspec(1),
    out_shape=jax.ShapeDtypeStruct((N_NODES, 1), F32),
)


def _local_indices(dsti, loci, base, row=0, n=BLK):
    for j in range(n // 16):
        d = dsti[row, pl.ds(j * 16, 16)]
        l = d - base
        ok = (l >= 0) & (l < HALF)
        loci[row, pl.ds(j * 16, 16)] = jnp.where(ok, l, DUMMY)


def _zero_chunks(sid, zeros_hbm, table):
    def zero_chunk(i, _):
        c = sid + i * NS

        @pl.when(c < NCH)
        def _():
            pltpu.sync_copy(zeros_hbm, table.at[pl.ds(c * ZCH, ZCH)])
        return 0

    lax.fori_loop(0, CH_ITERS, zero_chunk, 0)


def _copy_out_chunks(sid, base, table, out_hbm):
    def out_chunk(i, _):
        c = sid + i * NS

        @pl.when(c < NCH)
        def _():
            pltpu.sync_copy(table.at[pl.ds(c * ZCH, ZCH)],
                            out_hbm.at[pl.ds(base + c * ZCH, ZCH)])
        return 0

    lax.fori_loop(0, CH_ITERS, out_chunk, 0)


def _edge_compute(arows, brows, trows, g, bb):
    def edge2(e2, _):
        for u in range(2):
            e = e2 * 2 + u
            s0 = arows[e, pl.ds(0, 16)] + brows[e, pl.ds(0, 16)]
            s1 = arows[e, pl.ds(16, 16)] + brows[e, pl.ds(16, 16)]
            s2 = arows[e, pl.ds(32, 16)] + brows[e, pl.ds(32, 16)]
            s3 = arows[e, pl.ds(48, 16)] + brows[e, pl.ds(48, 16)]
            hs = jnp.sum(s0 + s1 + s2 + s3)
            hq = jnp.sum(s0 * s0 + s1 * s1 + s2 * s2 + s3 * s3)
            mean = hs * (1.0 / EMB)
            var = hq * (1.0 / EMB) - mean * mean
            x = jnp.full((16,), var + 1e-5, F32)
            iv = plsc.bitcast(x, jnp.int32)
            iv = 0x5F3759DF - lax.shift_right_logical(iv, 1)
            y = plsc.bitcast(iv, F32)
            xh = x * 0.5
            y = y * (1.5 - xh * y * y)
            y = y * (1.5 - xh * y * y)
            y = y * (1.5 - xh * y * y)
            c2 = mean * y
            for k, s in enumerate((s0, s1, s2, s3)):
                t = jnp.maximum((s * y - c2) * g[k] + bb[k], 0.0)
                trows[e, pl.ds(k * 16, 16)] = t
        return 0

    lax.fori_loop(0, BLK // 2, edge2, 0)


def _edge_body(a_hbm, b_hbm, di_hbm, zeros_hbm, gb_hbm, out_hbm,
               dsij0, loci0, dsij1, loci1, arows0, brows0, arows1, brows1,
               trows, gbv, table, semA0, semA1, semB0, semB1):
    cid = lax.axis_index("c")
    sid = lax.axis_index("s")
    base = cid * HALF

    pltpu.sync_copy(gb_hbm, gbv)
    _zero_chunks(sid, zeros_hbm, table)

    g = [gbv[0, pl.ds(k * 16, 16)] for k in range(4)]
    bb = [gbv[1, pl.ds(k * 16, 16)] for k in range(4)]

    plsc.subcore_barrier()

    sets = ((dsij0, loci0, arows0, brows0, semA0, semB0),
            (dsij1, loci1, arows1, brows1, semA1, semB1))

    def issue(k, st):
        dsij, _, arows, brows, semA, semB = st
        b = sid + k * NS
        pltpu.sync_copy(di_hbm.at[b], dsij)
        pltpu.async_copy(a_hbm.at[dsij.at[0]], arows, semA)
        pltpu.async_copy(b_hbm.at[dsij.at[1]], brows, semB)

    def process(st):
        dsij, loci, arows, brows, semA, semB = st
        _local_indices(dsij, loci, base)
        pltpu.make_async_copy(a_hbm.at[dsij.at[0]], arows, semA).wait()
        pltpu.make_async_copy(b_hbm.at[dsij.at[1]], brows, semB).wait()
        _edge_compute(arows, brows, trows, g, bb)
        pltpu.sync_copy(trows, table.at[loci.at[0]], add=True)

    issue(0, sets[0])

    def main_iter(i, _):
        k0 = 2 * i
        issue(k0 + 1, sets[1])
        process(sets[0])
        issue(k0 + 2, sets[0])
        process(sets[1])
        return 0

    lax.fori_loop(0, KMAIN // 2, main_iter, 0)

    process(sets[0])

    plsc.subcore_barrier()
    _copy_out_chunks(sid, base, table, out_hbm)


_edge_call = pl.kernel(
    _edge_body,
    out_type=jax.ShapeDtypeStruct((N_NODES, EMB), F32),
    mesh=plsc.VectorSubcoreMesh(core_axis_name="c", subcore_axis_name="s"),
    scratch_types=[
        pltpu.VMEM((2, BLK), jnp.int32),
        pltpu.VMEM((1, BLK), jnp.int32),
        pltpu.VMEM((2, BLK), jnp.int32),
        pltpu.VMEM((1, BLK), jnp.int32),
        pltpu.VMEM((BLK, EMB), F32),
        pltpu.VMEM((BLK, EMB), F32),
        pltpu.VMEM((BLK, EMB), F32),
        pltpu.VMEM((BLK, EMB), F32),
        pltpu.VMEM((BLK, EMB), F32),
        pltpu.VMEM((2, EMB), F32),
        pltpu.VMEM_SHARED((TROWS, EMB), F32),
        pltpu.SemaphoreType.DMA,
        pltpu.SemaphoreType.DMA,
        pltpu.SemaphoreType.DMA,
        pltpu.SemaphoreType.DMA,
    ],
    compiler_params=_SC_PARAMS,
)


BLKD = 640
DSUB = BLKD // BLK
NBLKD = N_EDGES // BLKD
KD = (NBLKD // NS) & ~1


def _deg_body(dst_hbm, zeros_hbm, out_hbm, dsti, loci, ones, table):
    cid = lax.axis_index("c")
    sid = lax.axis_index("s")
    base = cid * HALF

    _zero_chunks(sid, zeros_hbm, table)

    onehot = jnp.where(lax.iota(jnp.int32, 16) == 0, 1.0, 0.0).astype(F32)

    def init_row(r, _):
        ones[r, pl.ds(0, 16)] = onehot
        return 0

    lax.fori_loop(0, BLK, init_row, 0)

    plsc.subcore_barrier()

    def do_block(b):
        pltpu.sync_copy(dst_hbm.at[pl.ds(b, 1)], dsti)
        _local_indices(dsti, loci, base)
        pltpu.sync_copy(ones, table.at[loci.at[0]], add=True)

    def block_iter(i, _):
        b = sid + i * NS

        @pl.when(b < NBLK)
        def _():
            do_block(b)
        return 0

    lax.fori_loop(0, KMAIN + 1, block_iter, 0)

    plsc.subcore_barrier()
    _copy_out_chunks(sid, base, table, out_hbm)


_deg_call = pl.kernel(
    _deg_body,
    out_type=jax.ShapeDtypeStruct((N_NODES, DW), F32),
    mesh=plsc.VectorSubcoreMesh(core_axis_name="c", subcore_axis_name="s"),
    scratch_types=[
        pltpu.VMEM((1, BLK), jnp.int32),
        pltpu.VMEM((1, BLK), jnp.int32),
        pltpu.VMEM((BLK, DW), F32),
        pltpu.VMEM_SHARED((TROWS, DW), F32),
    ],
    compiler_params=_SC_PARAMS,
)


def kernel(constraint_features, edge_indices, edge_features, variable_features,
           params):
    p = params
    ei0 = edge_indices[0]
    ei1 = edge_indices[1]
    zeros64 = jnp.zeros((ZCH, EMB), F32)
    zeros16 = jnp.zeros((ZCH, DW), F32)
    r2 = lambda a: a.reshape(1, -1)

    c, v = _embed_call(
        constraint_features, variable_features,
        r2(p["cons_ln"]["g"]), r2(p["cons_ln"]["b"]),
        p["cons_l1"]["W"], r2(p["cons_l1"]["b"]),
        p["cons_l2"]["W"], r2(p["cons_l2"]["b"]),
        r2(p["var_ln"]["g"]), r2(p["var_ln"]["b"]),
        p["var_l1"]["W"], r2(p["var_l1"]["b"]),
        p["var_l2"]["W"], r2(p["var_l2"]["b"]),
    )

    b_edge = p["edge_ln"]["b"][0]

    def run_edge(pc, left, right, di):
        fe = b_edge * pc["fe"]["W"][0]
        A, B = _pre_call(right, left, pc["fl"]["W"],
                         r2(pc["fl"]["b"] + fe), pc["fr"]["W"])
        gb = jnp.stack([pc["ln1"]["g"], pc["ln1"]["b"]])
        return _edge_call(A, B, di, zeros64, gb)

    def run_post(pc, right, S, degt):
        return _post_call(
            S, degt, right, pc["ff"]["W"], r2(pc["ff"]["b"]),
            r2(pc["ln2"]["g"]), r2(pc["ln2"]["b"]),
            pc["o1"]["W"][:EMB], pc["o1"]["W"][EMB:], r2(pc["o1"]["b"]),
            pc["o2"]["W"], r2(pc["o2"]["b"]),
        )

    def run_conv(pc, left, right, di, degt):
        return run_post(pc, right, run_edge(pc, left, right, di), degt)

    e0b = ei0.reshape(-1, 1, BLK)
    e1b = ei1.reshape(-1, 1, BLK)
    di_vc = jnp.concatenate([e0b, e1b], 1)
    di_cv = jnp.concatenate([e1b, e0b], 1)

    S1 = run_edge(p["v2c"], v, c, di_vc)
    deg_c = _deg_call(ei0.reshape(-1, BLK), zeros16)
    deg_v = _deg_call(ei1.reshape(-1, BLK), zeros16)
    c = run_post(p["v2c"], c, S1, deg_c)

    v = run_conv(p["c2v"], c, v, di_cv, deg_v)
    c = run_conv(p["v2c2"], v, c, di_vc, deg_c)
    v = run_conv(p["c2v2"], c, v, di_cv, deg_v)

    out = _final_call(v, p["out_l1"]["W"], r2(p["out_l1"]["b"]),
                      p["out_l2"]["W"])
    return out[:, 0]
